# Initial kernel scaffold; baseline (speedup 1.0000x reference)
import jax
import jax.numpy as jnp
from jax.experimental import pallas as pl


def kernel(boxes, scores, labels):
    raise NotImplementedError("write your pallas kernel here")


# speedup vs baseline: 158.1136x; 158.1136x over previous
import jax
import jax.numpy as jnp
from jax.experimental import pallas as pl

_N = 20000
_B = 512
_NB = 40
_NP = _B * _NB
_IOU_T = 0.5


def _pair_iou(cx1, cy1, cx2, cy2, ca, rx1, ry1, rx2, ry2, ra):
    xx1 = jnp.maximum(cx1, rx1)
    yy1 = jnp.maximum(cy1, ry1)
    xx2 = jnp.minimum(cx2, rx2)
    yy2 = jnp.minimum(cy2, ry2)
    inter = jnp.maximum(xx2 - xx1, 0.0) * jnp.maximum(yy2 - yy1, 0.0)
    return inter / (ca + ra - inter + 1e-9)


def _nms_kernel(x1_ref, y1_ref, x2_ref, y2_ref, keep_ref):
    row_i = jax.lax.broadcasted_iota(jnp.int32, (_B, _B), 0)
    col_i = jax.lax.broadcasted_iota(jnp.int32, (_B, _B), 1)
    eyef = (row_i == col_i).astype(jnp.float32)
    ltf = (col_i < row_i).astype(jnp.float32)
    utf = (row_i < col_i).astype(jnp.float32)

    def to_col(row):
        return jnp.sum(eyef * row, axis=1, keepdims=True)

    def to_row(col):
        return jnp.sum(eyef * col, axis=0, keepdims=True)

    def outer(j, carry):
        x1r = x1_ref[j]
        y1r = y1_ref[j]
        x2r = x2_ref[j]
        y2r = y2_ref[j]
        cx1 = to_col(x1r)
        cy1 = to_col(y1r)
        cx2 = to_col(x2r)
        cy2 = to_col(y2r)
        ca = (cx2 - cx1) * (cy2 - cy1)

        def inner(i, supp):
            kx1 = x1_ref[i]
            ky1 = y1_ref[i]
            kx2 = x2_ref[i]
            ky2 = y2_ref[i]
            ka = (kx2 - kx1) * (ky2 - ky1)
            kkeep = keep_ref[i]
            iou = _pair_iou(cx1, cy1, cx2, cy2, ca, kx1, ky1, kx2, ky2, ka)
            hit = jnp.where(iou > _IOU_T, 1.0, 0.0) * kkeep
            return jnp.maximum(supp, jnp.max(hit, axis=1, keepdims=True))

        supp = jax.lax.fori_loop(0, j, inner, jnp.zeros((_B, 1), jnp.float32))
        p_col = 1.0 - supp
        p_row = to_row(p_col)

        ra = (x2r - x1r) * (y2r - y1r)
        iou_s = _pair_iou(cx1, cy1, cx2, cy2, ca, x1r, y1r, x2r, y2r, ra)
        sgt = jnp.where(iou_s > _IOU_T, 1.0, 0.0)
        mat_a = sgt * ltf
        mat_b = sgt * utf

        def cond(carry):
            return carry[1]

        def body(carry):
            krow, _ = carry
            s1 = jnp.max(mat_a * krow, axis=1, keepdims=True)
            kcol = p_col * (1.0 - s1)
            s2 = jnp.max(mat_b * kcol, axis=0, keepdims=True)
            newrow = p_row * (1.0 - s2)
            changed = jnp.sum(jnp.abs(newrow - krow)) > 0.0
            return (newrow, changed)

        krow, _ = jax.lax.while_loop(cond, body, (p_row, jnp.bool_(True)))
        keep_ref[j] = krow
        return carry

    jax.lax.fori_loop(0, _NB, outer, 0)


def kernel(boxes, scores, labels):
    order = jnp.argsort(-scores)
    boxes_s = boxes[order]
    scores_s = scores[order]
    labels_s = labels[order]

    pad = _NP - _N
    x1 = jnp.pad(boxes_s[:, 0], (0, pad)).reshape(_NB, 1, _B)
    y1 = jnp.pad(boxes_s[:, 1], (0, pad)).reshape(_NB, 1, _B)
    x2 = jnp.pad(boxes_s[:, 2], (0, pad)).reshape(_NB, 1, _B)
    y2 = jnp.pad(boxes_s[:, 3], (0, pad)).reshape(_NB, 1, _B)

    keepf = pl.pallas_call(
        _nms_kernel,
        out_shape=jax.ShapeDtypeStruct((_NB, 1, _B), jnp.float32),
    )(x1, y1, x2, y2)

    keepf = keepf.reshape(_NP)[:_N]
    keep = keepf > 0.5
    boxes_out = boxes_s * keepf[:, None]
    scores_out = scores_s * keepf
    labels_out = labels_s * keep.astype(labels_s.dtype)
    return (boxes_out, scores_out, labels_out, keep)


# speedup vs baseline: 180.8094x; 1.1435x over previous
import jax
import jax.numpy as jnp
from jax.experimental import pallas as pl

_N = 20000
_B = 512
_NB = 40
_NP = _B * _NB
_IOU_T = 0.5


def _pair_iou(cx1, cy1, cx2, cy2, ca, rx1, ry1, rx2, ry2, ra):
    xx1 = jnp.maximum(cx1, rx1)
    yy1 = jnp.maximum(cy1, ry1)
    xx2 = jnp.minimum(cx2, rx2)
    yy2 = jnp.minimum(cy2, ry2)
    inter = jnp.maximum(xx2 - xx1, 0.0) * jnp.maximum(yy2 - yy1, 0.0)
    return inter / (ca + ra - inter + 1e-9)


def _nms_kernel(x1_ref, y1_ref, x2_ref, y2_ref, keep_ref):
    row_i = jax.lax.broadcasted_iota(jnp.int32, (_B, _B), 0)
    col_i = jax.lax.broadcasted_iota(jnp.int32, (_B, _B), 1)
    eyef = (row_i == col_i).astype(jnp.float32)
    ltf = (col_i < row_i).astype(jnp.float32)
    utf = (row_i < col_i).astype(jnp.float32)

    def to_col(row):
        return jnp.sum(eyef * row, axis=1, keepdims=True)

    def to_row(col):
        return jnp.sum(eyef * col, axis=0, keepdims=True)

    def outer(j, carry):
        x1r = x1_ref[j]
        y1r = y1_ref[j]
        x2r = x2_ref[j]
        y2r = y2_ref[j]
        cx1 = to_col(x1r)
        cy1 = to_col(y1r)
        cx2 = to_col(x2r)
        cy2 = to_col(y2r)
        ca = (cx2 - cx1) * (cy2 - cy1)

        def inner(i, supp):
            dead = keep_ref[i] < 0.5
            kx1 = jnp.where(dead, 1e9, x1_ref[i])
            ky1 = jnp.where(dead, 1e9, y1_ref[i])
            kx2 = jnp.where(dead, 1e9, x2_ref[i])
            ky2 = jnp.where(dead, 1e9, y2_ref[i])
            ka = (kx2 - kx1) * (ky2 - ky1)
            iou = _pair_iou(cx1, cy1, cx2, cy2, ca, kx1, ky1, kx2, ky2, ka)
            return jnp.maximum(supp, jnp.max(iou, axis=1, keepdims=True))

        supp = jax.lax.fori_loop(0, j, inner, jnp.zeros((_B, 1), jnp.float32))
        p_col = jnp.where(supp > _IOU_T, 0.0, 1.0)
        p_row = to_row(p_col)

        ra = (x2r - x1r) * (y2r - y1r)
        iou_s = _pair_iou(cx1, cy1, cx2, cy2, ca, x1r, y1r, x2r, y2r, ra)
        sgt = jnp.where(iou_s > _IOU_T, 1.0, 0.0)
        mat_a = sgt * ltf
        mat_b = sgt * utf

        def cond(carry):
            return carry[1]

        def body(carry):
            krow, _ = carry
            s1 = jnp.max(mat_a * krow, axis=1, keepdims=True)
            kcol = p_col * (1.0 - s1)
            s2 = jnp.max(mat_b * kcol, axis=0, keepdims=True)
            newrow = p_row * (1.0 - s2)
            changed = jnp.sum(jnp.abs(newrow - krow)) > 0.0
            return (newrow, changed)

        krow, _ = jax.lax.while_loop(cond, body, (p_row, jnp.bool_(True)))
        keep_ref[j] = krow
        return carry

    jax.lax.fori_loop(0, _NB, outer, 0)


def kernel(boxes, scores, labels):
    order = jnp.argsort(-scores)
    boxes_s = boxes[order]
    scores_s = scores[order]
    labels_s = labels[order]

    pad = _NP - _N
    x1 = jnp.pad(boxes_s[:, 0], (0, pad)).reshape(_NB, 1, _B)
    y1 = jnp.pad(boxes_s[:, 1], (0, pad)).reshape(_NB, 1, _B)
    x2 = jnp.pad(boxes_s[:, 2], (0, pad)).reshape(_NB, 1, _B)
    y2 = jnp.pad(boxes_s[:, 3], (0, pad)).reshape(_NB, 1, _B)

    keepf = pl.pallas_call(
        _nms_kernel,
        out_shape=jax.ShapeDtypeStruct((_NB, 1, _B), jnp.float32),
    )(x1, y1, x2, y2)

    keepf = keepf.reshape(_NP)[:_N]
    keep = keepf > 0.5
    boxes_out = boxes_s * keepf[:, None]
    scores_out = scores_s * keepf
    labels_out = labels_s * keep.astype(labels_s.dtype)
    return (boxes_out, scores_out, labels_out, keep)


# speedup vs baseline: 185.4343x; 1.0256x over previous
import jax
import jax.numpy as jnp
from jax.experimental import pallas as pl

_N = 20000
_B = 1024
_NB = 20
_NP = _B * _NB
_IOU_T = 0.5


def _pair_iou(cx1, cy1, cx2, cy2, ca, rx1, ry1, rx2, ry2, ra):
    xx1 = jnp.maximum(cx1, rx1)
    yy1 = jnp.maximum(cy1, ry1)
    xx2 = jnp.minimum(cx2, rx2)
    yy2 = jnp.minimum(cy2, ry2)
    inter = jnp.maximum(xx2 - xx1, 0.0) * jnp.maximum(yy2 - yy1, 0.0)
    return inter / (ca + ra - inter + 1e-9)


def _nms_kernel(x1_ref, y1_ref, x2_ref, y2_ref, keep_ref):
    row_i = jax.lax.broadcasted_iota(jnp.int32, (_B, _B), 0)
    col_i = jax.lax.broadcasted_iota(jnp.int32, (_B, _B), 1)
    eyef = (row_i == col_i).astype(jnp.float32)
    ltf = (col_i < row_i).astype(jnp.float32)
    utf = (row_i < col_i).astype(jnp.float32)

    def to_col(row):
        return jnp.sum(eyef * row, axis=1, keepdims=True)

    def to_row(col):
        return jnp.sum(eyef * col, axis=0, keepdims=True)

    def outer(j, carry):
        x1r = x1_ref[j]
        y1r = y1_ref[j]
        x2r = x2_ref[j]
        y2r = y2_ref[j]
        cx1 = to_col(x1r)
        cy1 = to_col(y1r)
        cx2 = to_col(x2r)
        cy2 = to_col(y2r)
        ca = (cx2 - cx1) * (cy2 - cy1)

        def inner(i, supp):
            dead = keep_ref[i] < 0.5
            kx1 = jnp.where(dead, 1e9, x1_ref[i])
            ky1 = jnp.where(dead, 1e9, y1_ref[i])
            kx2 = jnp.where(dead, 1e9, x2_ref[i])
            ky2 = jnp.where(dead, 1e9, y2_ref[i])
            ka = (kx2 - kx1) * (ky2 - ky1)
            iou = _pair_iou(cx1, cy1, cx2, cy2, ca, kx1, ky1, kx2, ky2, ka)
            return jnp.maximum(supp, jnp.max(iou, axis=1, keepdims=True))

        supp = jax.lax.fori_loop(0, j, inner, jnp.zeros((_B, 1), jnp.float32))
        p_col = jnp.where(supp > _IOU_T, 0.0, 1.0)
        p_row = to_row(p_col)

        ra = (x2r - x1r) * (y2r - y1r)
        iou_s = _pair_iou(cx1, cy1, cx2, cy2, ca, x1r, y1r, x2r, y2r, ra)
        sgt = jnp.where(iou_s > _IOU_T, 1.0, 0.0)
        mat_a = sgt * ltf
        mat_b = sgt * utf

        def cond(carry):
            return carry[1]

        def body(carry):
            krow, _ = carry
            s1 = jnp.max(mat_a * krow, axis=1, keepdims=True)
            kcol = p_col * (1.0 - s1)
            s2 = jnp.max(mat_b * kcol, axis=0, keepdims=True)
            newrow = p_row * (1.0 - s2)
            changed = jnp.sum(jnp.abs(newrow - krow)) > 0.0
            return (newrow, changed)

        krow, _ = jax.lax.while_loop(cond, body, (p_row, jnp.bool_(True)))
        keep_ref[j] = krow
        return carry

    jax.lax.fori_loop(0, _NB, outer, 0)


def kernel(boxes, scores, labels):
    order = jnp.argsort(-scores)
    boxes_s = boxes[order]
    scores_s = scores[order]
    labels_s = labels[order]

    pad = _NP - _N
    x1 = jnp.pad(boxes_s[:, 0], (0, pad)).reshape(_NB, 1, _B)
    y1 = jnp.pad(boxes_s[:, 1], (0, pad)).reshape(_NB, 1, _B)
    x2 = jnp.pad(boxes_s[:, 2], (0, pad)).reshape(_NB, 1, _B)
    y2 = jnp.pad(boxes_s[:, 3], (0, pad)).reshape(_NB, 1, _B)

    keepf = pl.pallas_call(
        _nms_kernel,
        out_shape=jax.ShapeDtypeStruct((_NB, 1, _B), jnp.float32),
    )(x1, y1, x2, y2)

    keepf = keepf.reshape(_NP)[:_N]
    keep = keepf > 0.5
    boxes_out = boxes_s * keepf[:, None]
    scores_out = scores_s * keepf
    labels_out = labels_s * keep.astype(labels_s.dtype)
    return (boxes_out, scores_out, labels_out, keep)
